# baseline (speedup 1.0000x reference)
import functools

import jax
import jax.numpy as jnp
from jax import lax
from jax.experimental import pallas as pl
from jax.experimental.pallas import tpu as pltpu
from jax.experimental.pallas import tpu_sc as plsc

NC, NS, L = 2, 16, 16
NW = NC * NS
VOCAB = 1000000
D = 64
SEQ = 200
N_ROWS = 4096 * SEQ
ROWS_PER_W = N_ROWS // NW
CHUNK = 512
IDXW = 128
N_CHUNKS = ROWS_PER_W // CHUNK
EPS = 1e-5


def _rsqrt(x):
    i = lax.bitcast_convert_type(x, jnp.int32)
    i = jnp.int32(0x5F3759DF) - lax.shift_right_logical(i, 1)
    y = lax.bitcast_convert_type(i, jnp.float32)
    for _ in range(3):
        y = y * (1.5 - 0.5 * x * y * y)
    return y


def _body(x_ref, tok_ref, pos_ref, gam_ref, bet_ref, out_ref,
          idx_v, rows_v, pos_v, gam_v, bet_v, gsem):
    wid = lax.axis_index("s") * NC + lax.axis_index("c")

    pltpu.sync_copy(pos_ref, pos_v)
    pltpu.sync_copy(gam_ref, gam_v)
    pltpu.sync_copy(bet_ref, bet_v)

    g = [gam_v[pl.ds(16 * i, 16)] for i in range(4)]
    b = [bet_v[pl.ds(16 * i, 16)] for i in range(4)]

    @pl.loop(0, N_CHUNKS)
    def chunk_loop(k):
        crow = wid * (ROWS_PER_W // IDXW) + k * (CHUNK // IDXW)
        pltpu.sync_copy(x_ref.at[pl.ds(crow, CHUNK // IDXW)], idx_v)
        for j in range(CHUNK // IDXW):
            pltpu.async_copy(tok_ref.at[idx_v.at[j]],
                             rows_v.at[pl.ds(j * IDXW, IDXW)], gsem)
        for j in range(CHUNK // IDXW):
            pltpu.make_async_copy(tok_ref.at[idx_v.at[j]],
                                  rows_v.at[pl.ds(j * IDXW, IDXW)], gsem).wait()

        base200 = lax.rem(k * CHUNK, SEQ)

        @pl.loop(0, CHUNK)
        def row_loop(r):
            s = lax.rem(base200 + r, SEQ)
            e0 = rows_v[r, pl.ds(0, 16)] + pos_v[s, pl.ds(0, 16)]
            e1 = rows_v[r, pl.ds(16, 16)] + pos_v[s, pl.ds(16, 16)]
            e2 = rows_v[r, pl.ds(32, 16)] + pos_v[s, pl.ds(32, 16)]
            e3 = rows_v[r, pl.ds(48, 16)] + pos_v[s, pl.ds(48, 16)]
            t = (e0 + e1) + (e2 + e3)
            mean = jnp.sum(t) * (1.0 / D)
            d0 = e0 - mean
            d1 = e1 - mean
            d2 = e2 - mean
            d3 = e3 - mean
            sq = (d0 * d0 + d1 * d1) + (d2 * d2 + d3 * d3)
            var = jnp.sum(sq) * (1.0 / D)
            rs = _rsqrt(var + EPS)
            rows_v[r, pl.ds(0, 16)] = (d0 * rs) * g[0] + b[0]
            rows_v[r, pl.ds(16, 16)] = (d1 * rs) * g[1] + b[1]
            rows_v[r, pl.ds(32, 16)] = (d2 * rs) * g[2] + b[2]
            rows_v[r, pl.ds(48, 16)] = (d3 * rs) * g[3] + b[3]

        pltpu.sync_copy(rows_v,
                        out_ref.at[pl.ds(wid * ROWS_PER_W + k * CHUNK, CHUNK)])


@jax.jit
def _emb(x2d, tok_table, pos_table, gamma, beta):
    mesh = plsc.VectorSubcoreMesh(core_axis_name="c", subcore_axis_name="s")
    run = pl.kernel(
        _body,
        out_type=jax.ShapeDtypeStruct((N_ROWS, D), jnp.float32),
        mesh=mesh,
        compiler_params=pltpu.CompilerParams(
            needs_layout_passes=False, use_tc_tiling_on_sc=False),
        scratch_types=[
            pltpu.VMEM((CHUNK // IDXW, IDXW), jnp.int32),
            pltpu.VMEM((CHUNK, D), jnp.float32),
            pltpu.VMEM((SEQ, D), jnp.float32),
            pltpu.VMEM((D,), jnp.float32),
            pltpu.VMEM((D,), jnp.float32),
            pltpu.SemaphoreType.DMA,
        ],
    )
    return run(x2d, tok_table, pos_table, gamma, beta)


def kernel(x, tok_table, pos_table, gamma, beta):
    batch, seq = x.shape
    x2d = x.astype(jnp.int32).reshape(N_ROWS // IDXW, IDXW)
    out = _emb(x2d, tok_table, pos_table, gamma, beta)
    return out.reshape(batch, seq, D)


# speedup vs baseline: 1.8199x; 1.8199x over previous
import jax
import jax.numpy as jnp
from jax import lax
from jax.experimental import pallas as pl
from jax.experimental.pallas import tpu as pltpu
from jax.experimental.pallas import tpu_sc as plsc

NC, NS = 2, 16
NW = NC * NS
D = 64
SEQ = 200
N_ROWS = 4096 * SEQ
ROWS_PER_W = N_ROWS // NW
CHUNK = 512
IDXW = 128
N_CHUNKS = ROWS_PER_W // CHUNK
IDX_ROWS_W = ROWS_PER_W // IDXW
EPS = 1e-5


def _body(x_ref, tok_ref, pos_ref, gam_ref, bet_ref, out_ref,
          idx_all, buf_a, buf_b, pos_v, gam_v, bet_v, gsem, osem):
    wid = lax.axis_index("s") * NC + lax.axis_index("c")
    out_base = wid * ROWS_PER_W

    pltpu.sync_copy(pos_ref, pos_v)
    pltpu.sync_copy(gam_ref, gam_v)
    pltpu.sync_copy(bet_ref, bet_v)
    pltpu.sync_copy(x_ref.at[pl.ds(wid * IDX_ROWS_W, IDX_ROWS_W)], idx_all)

    g = [gam_v[pl.ds(16 * i, 16)] for i in range(4)]
    b = [bet_v[pl.ds(16 * i, 16)] for i in range(4)]

    def fire(c, buf):
        for j in range(CHUNK // IDXW):
            pltpu.async_copy(tok_ref.at[idx_all.at[c * (CHUNK // IDXW) + j]],
                             buf.at[pl.ds(j * IDXW, IDXW)], gsem)

    def wait_gather(c, buf):
        for j in range(CHUNK // IDXW):
            pltpu.make_async_copy(
                tok_ref.at[idx_all.at[c * (CHUNK // IDXW) + j]],
                buf.at[pl.ds(j * IDXW, IDXW)], gsem).wait()

    def wait_wb():
        pltpu.make_async_copy(
            buf_a, out_ref.at[pl.ds(out_base, CHUNK)], osem).wait()

    def compute(buf, c):
        base200 = lax.rem(c * CHUNK, SEQ)

        @plsc.parallel_loop(0, CHUNK, unroll=8)
        def row_loop(r):
            s = lax.rem(base200 + r, SEQ)
            e0 = buf[r, pl.ds(0, 16)] + pos_v[s, pl.ds(0, 16)]
            e1 = buf[r, pl.ds(16, 16)] + pos_v[s, pl.ds(16, 16)]
            e2 = buf[r, pl.ds(32, 16)] + pos_v[s, pl.ds(32, 16)]
            e3 = buf[r, pl.ds(48, 16)] + pos_v[s, pl.ds(48, 16)]
            t = (e0 + e1) + (e2 + e3)
            mean = jnp.sum(t) * (1.0 / D)
            d0 = e0 - mean
            d1 = e1 - mean
            d2 = e2 - mean
            d3 = e3 - mean
            sq = (d0 * d0 + d1 * d1) + (d2 * d2 + d3 * d3)
            var = jnp.sum(sq) * (1.0 / D)
            x = var + EPS
            i = lax.bitcast_convert_type(x, jnp.int32)
            i = jnp.int32(0x5F3759DF) - lax.shift_right_logical(i, 1)
            y = lax.bitcast_convert_type(i, jnp.float32)
            y = y * (1.5 - 0.5 * x * y * y)
            rs = y * (1.5 - 0.5 * x * y * y)
            buf[r, pl.ds(0, 16)] = (d0 * rs) * g[0] + b[0]
            buf[r, pl.ds(16, 16)] = (d1 * rs) * g[1] + b[1]
            buf[r, pl.ds(32, 16)] = (d2 * rs) * g[2] + b[2]
            buf[r, pl.ds(48, 16)] = (d3 * rs) * g[3] + b[3]

    def writeback(buf, c):
        pltpu.async_copy(buf, out_ref.at[pl.ds(out_base + c * CHUNK, CHUNK)],
                         osem)

    fire(0, buf_a)

    @pl.loop(0, N_CHUNKS // 2)
    def pair(j):
        ca = 2 * j
        cb = 2 * j + 1

        @pl.when(j > 0)
        def _():
            wait_wb()
        fire(cb, buf_b)
        wait_gather(ca, buf_a)
        compute(buf_a, ca)
        writeback(buf_a, ca)

        wait_wb()

        @pl.when(j < N_CHUNKS // 2 - 1)
        def _():
            fire(cb + 1, buf_a)
        wait_gather(cb, buf_b)
        compute(buf_b, cb)
        writeback(buf_b, cb)

    wait_wb()


@jax.jit
def _emb(x2d, tok_table, pos_table, gamma, beta):
    mesh = plsc.VectorSubcoreMesh(core_axis_name="c", subcore_axis_name="s")
    run = pl.kernel(
        _body,
        out_type=jax.ShapeDtypeStruct((N_ROWS, D), jnp.float32),
        mesh=mesh,
        compiler_params=pltpu.CompilerParams(
            needs_layout_passes=False, use_tc_tiling_on_sc=False),
        scratch_types=[
            pltpu.VMEM((IDX_ROWS_W, IDXW), jnp.int32),
            pltpu.VMEM((CHUNK, D), jnp.float32),
            pltpu.VMEM((CHUNK, D), jnp.float32),
            pltpu.VMEM((SEQ, D), jnp.float32),
            pltpu.VMEM((D,), jnp.float32),
            pltpu.VMEM((D,), jnp.float32),
            pltpu.SemaphoreType.DMA,
            pltpu.SemaphoreType.DMA,
        ],
    )
    return run(x2d, tok_table, pos_table, gamma, beta)


def kernel(x, tok_table, pos_table, gamma, beta):
    batch, seq = x.shape
    x2d = x.astype(jnp.int32).reshape(N_ROWS // IDXW, IDXW)
    out = _emb(x2d, tok_table, pos_table, gamma, beta)
    return out.reshape(batch, seq, D)


# speedup vs baseline: 1.8304x; 1.0057x over previous
import jax
import jax.numpy as jnp
from jax import lax
from jax.experimental import pallas as pl
from jax.experimental.pallas import tpu as pltpu
from jax.experimental.pallas import tpu_sc as plsc

NC, NS = 2, 16
NW = NC * NS
D = 64
SEQ = 200
BATCH = 4096
N_ROWS = BATCH * SEQ
ROWS_PER_W = N_ROWS // NW
SEQ_PER_W = BATCH // NW
CHUNK = 2 * SEQ
IDXW = 80
N_GAT = CHUNK // IDXW
N_CHUNKS = ROWS_PER_W // CHUNK
IDX_ROWS_W = ROWS_PER_W // IDXW
EPS = 1e-5


def _body(x_ref, tok_ref, pos_ref, gam_ref, bet_ref, out_ref,
          idx_all, buf_a, buf_b, pos_v, gam_v, bet_v, gsem, osem):
    wid = lax.axis_index("s") * NC + lax.axis_index("c")
    seq_base = wid * SEQ_PER_W

    pltpu.sync_copy(pos_ref, pos_v)
    pltpu.sync_copy(gam_ref, gam_v)
    pltpu.sync_copy(bet_ref, bet_v)
    pltpu.sync_copy(x_ref.at[pl.ds(wid * IDX_ROWS_W, IDX_ROWS_W)], idx_all)

    g = [gam_v[pl.ds(16 * i, 16)] for i in range(4)]
    b = [bet_v[pl.ds(16 * i, 16)] for i in range(4)]

    def fire(c, buf):
        for j in range(N_GAT):
            pltpu.async_copy(tok_ref.at[idx_all.at[c * N_GAT + j]],
                             buf.at[pl.ds(j * IDXW, IDXW)], gsem)

    def wait_gather(c, buf):
        for j in range(N_GAT):
            pltpu.make_async_copy(
                tok_ref.at[idx_all.at[c * N_GAT + j]],
                buf.at[pl.ds(j * IDXW, IDXW)], gsem).wait()

    def writeback(buf, c):
        pltpu.async_copy(buf.at[pl.ds(0, SEQ)],
                         out_ref.at[seq_base + 2 * c], osem)
        pltpu.async_copy(buf.at[pl.ds(SEQ, SEQ)],
                         out_ref.at[seq_base + 2 * c + 1], osem)

    def wait_wb():
        pltpu.make_async_copy(buf_a.at[pl.ds(0, SEQ)],
                              out_ref.at[seq_base], osem).wait()
        pltpu.make_async_copy(buf_a.at[pl.ds(0, SEQ)],
                              out_ref.at[seq_base], osem).wait()

    def norm_row(e0, e1, e2, e3):
        t = (e0 + e1) + (e2 + e3)
        mean = jnp.sum(t) * (1.0 / D)
        d0 = e0 - mean
        d1 = e1 - mean
        d2 = e2 - mean
        d3 = e3 - mean
        sq = (d0 * d0 + d1 * d1) + (d2 * d2 + d3 * d3)
        var = jnp.sum(sq) * (1.0 / D)
        x = var + EPS
        i = lax.bitcast_convert_type(x, jnp.int32)
        i = jnp.int32(0x5F3759DF) - lax.shift_right_logical(i, 1)
        y = lax.bitcast_convert_type(i, jnp.float32)
        y = y * (1.5 - 0.5 * x * y * y)
        rs = y * (1.5 - 0.5 * x * y * y)
        return [(d0 * rs) * g[0] + b[0], (d1 * rs) * g[1] + b[1],
                (d2 * rs) * g[2] + b[2], (d3 * rs) * g[3] + b[3]]

    def compute(buf):
        @plsc.parallel_loop(0, SEQ, unroll=8)
        def row_loop(s):
            p = [pos_v[s, pl.ds(16 * i, 16)] for i in range(4)]
            ea = [buf[s, pl.ds(16 * i, 16)] + p[i] for i in range(4)]
            eb = [buf[SEQ + s, pl.ds(16 * i, 16)] + p[i] for i in range(4)]
            oa = norm_row(*ea)
            ob = norm_row(*eb)
            for i in range(4):
                buf[s, pl.ds(16 * i, 16)] = oa[i]
                buf[SEQ + s, pl.ds(16 * i, 16)] = ob[i]

    fire(0, buf_a)

    @pl.loop(0, N_CHUNKS // 2)
    def pair(j):
        ca = 2 * j
        cb = 2 * j + 1

        @pl.when(j > 0)
        def _():
            wait_wb()
        fire(cb, buf_b)
        wait_gather(ca, buf_a)
        compute(buf_a)
        writeback(buf_a, ca)

        wait_wb()

        @pl.when(j < N_CHUNKS // 2 - 1)
        def _():
            fire(cb + 1, buf_a)
        wait_gather(cb, buf_b)
        compute(buf_b)
        writeback(buf_b, cb)

    wait_wb()


@jax.jit
def _emb(x2d, tok_table, pos_table, gamma, beta):
    mesh = plsc.VectorSubcoreMesh(core_axis_name="c", subcore_axis_name="s")
    run = pl.kernel(
        _body,
        out_type=jax.ShapeDtypeStruct((BATCH, SEQ, D), jnp.float32),
        mesh=mesh,
        compiler_params=pltpu.CompilerParams(
            needs_layout_passes=False, use_tc_tiling_on_sc=False),
        scratch_types=[
            pltpu.VMEM((IDX_ROWS_W, IDXW), jnp.int32),
            pltpu.VMEM((CHUNK, D), jnp.float32),
            pltpu.VMEM((CHUNK, D), jnp.float32),
            pltpu.VMEM((SEQ, D), jnp.float32),
            pltpu.VMEM((D,), jnp.float32),
            pltpu.VMEM((D,), jnp.float32),
            pltpu.SemaphoreType.DMA,
            pltpu.SemaphoreType.DMA,
        ],
    )
    return run(x2d, tok_table, pos_table, gamma, beta)


def kernel(x, tok_table, pos_table, gamma, beta):
    x2d = x.astype(jnp.int32).reshape(N_ROWS // IDXW, IDXW)
    return _emb(x2d, tok_table, pos_table, gamma, beta)
